# baseline (speedup 1.0000x reference)
import functools

import jax
import jax.numpy as jnp
from jax import lax
from jax.experimental import pallas as pl
from jax.experimental.pallas import tpu as pltpu
from jax.experimental.pallas import tpu_sc as plsc

_BATCH = 4096
_HIST = 50
_HIST_PAD = 56
_D = 64
_D_PAD = 128
_NC = 2
_NS = 16
_NW = _NC * _NS
_ROWS_PER_W = _BATCH // _NW
_CROWS = 8
_NCHUNK = _ROWS_PER_W // _CROWS
_NBUF = 3

_mesh = plsc.VectorSubcoreMesh(core_axis_name="c", subcore_axis_name="s")


@functools.partial(
    pl.kernel,
    mesh=_mesh,
    out_type=jax.ShapeDtypeStruct((_BATCH, _HIST_PAD, _D_PAD), jnp.float32),
    scratch_types=[
        pltpu.VMEM((_ROWS_PER_W * _HIST,), jnp.int32),
        pltpu.VMEM((_NBUF, _CROWS * _HIST, _D), jnp.float32),
        pltpu.SemaphoreType.DMA,
        pltpu.SemaphoreType.DMA,
        pltpu.SemaphoreType.DMA,
        pltpu.SemaphoreType.DMA,
        pltpu.SemaphoreType.DMA,
        pltpu.SemaphoreType.DMA,
    ],
    compiler_params=pltpu.CompilerParams(use_tc_tiling_on_sc=False),
)
def _sc_gather(idx_hbm, table_hbm, out_hbm, idx_v, rows_v,
               g0, g1, g2, o0, o1, o2):
    wid = lax.axis_index("s") * _NC + lax.axis_index("c")
    base = wid * _ROWS_PER_W
    gsem = (g0, g1, g2)
    osem = (o0, o1, o2)
    pltpu.sync_copy(idx_hbm.at[wid], idx_v)

    def start_gather(c):
        idx_chunk = idx_v.at[pl.ds(c * _CROWS * _HIST, _CROWS * _HIST)]
        return pltpu.async_copy(table_hbm.at[idx_chunk], rows_v.at[c % _NBUF],
                                gsem[c % _NBUF])

    def start_writes(c):
        b = c % _NBUF
        return [
            pltpu.async_copy(
                rows_v.at[b].at[pl.ds(r * _HIST, _HIST)],
                out_hbm.at[base + c * _CROWS + r].at[pl.ds(0, _HIST),
                                                     pl.ds(0, _D)],
                osem[b])
            for r in range(_CROWS)
        ]

    gather = [None] * _NCHUNK
    write = [None] * _NCHUNK
    for c in range(_NBUF - 1):
        gather[c] = start_gather(c)
    for c in range(_NCHUNK):
        cn = c + _NBUF - 1
        if cn < _NCHUNK:
            prev = cn - _NBUF
            if prev >= 0:
                for h in write[prev]:
                    h.wait()
            gather[cn] = start_gather(cn)
        gather[c].wait()
        write[c] = start_writes(c)
    for c in range(_NCHUNK - _NBUF, _NCHUNK):
        if c >= 0:
            for h in write[c]:
                h.wait()


_TB = 256


def _tc_transpose_body(x_ref, o_ref):
    for t in range(_HIST):
        o_ref[t] = jnp.transpose(x_ref[:, t, :], (1, 0))


_tc_transpose = pl.pallas_call(
    _tc_transpose_body,
    grid=(_BATCH // _TB,),
    in_specs=[pl.BlockSpec((_TB, _HIST, _D), lambda i: (i, 0, 0))],
    out_specs=pl.BlockSpec((_HIST, _D, _TB), lambda i: (0, 0, i)),
    out_shape=jax.ShapeDtypeStruct((_HIST, _D, _BATCH), jnp.float32),
)


def kernel(inputs, emb_table):
    idx = inputs.astype(jnp.int32).reshape(_NW, _ROWS_PER_W * _HIST)
    padded = _sc_gather(idx, emb_table)
    out_t = _tc_transpose(padded[:, :_HIST, :_D])
    return out_t.transpose(2, 0, 1)


# speedup vs baseline: 1.0503x; 1.0503x over previous
import functools

import jax
import jax.numpy as jnp
from jax import lax
from jax.experimental import pallas as pl
from jax.experimental.pallas import tpu as pltpu
from jax.experimental.pallas import tpu_sc as plsc

_BATCH = 4096
_HIST = 50
_HIST_PAD = 56
_D = 64
_D_PAD = 128
_NC = 2
_NS = 16
_NW = _NC * _NS
_ROWS_PER_W = _BATCH // _NW
_CROWS = 8
_NCHUNK = _ROWS_PER_W // _CROWS
_NBUF = 3

_mesh = plsc.VectorSubcoreMesh(core_axis_name="c", subcore_axis_name="s")


@functools.partial(
    pl.kernel,
    mesh=_mesh,
    out_type=jax.ShapeDtypeStruct((_BATCH, _HIST_PAD, _D_PAD), jnp.float32),
    scratch_types=[
        pltpu.VMEM((_ROWS_PER_W * _HIST,), jnp.int32),
        pltpu.VMEM((_NBUF, _CROWS * _HIST, _D), jnp.float32),
        pltpu.SemaphoreType.DMA,
        pltpu.SemaphoreType.DMA,
        pltpu.SemaphoreType.DMA,
        pltpu.SemaphoreType.DMA,
        pltpu.SemaphoreType.DMA,
        pltpu.SemaphoreType.DMA,
    ],
    compiler_params=pltpu.CompilerParams(use_tc_tiling_on_sc=False),
)
def _sc_gather(idx_hbm, table_hbm, out_hbm, idx_v, rows_v,
               g0, g1, g2, o0, o1, o2):
    wid = lax.axis_index("s") * _NC + lax.axis_index("c")
    base = wid * _ROWS_PER_W
    gsem = (g0, g1, g2)
    osem = (o0, o1, o2)
    pltpu.sync_copy(idx_hbm.at[wid], idx_v)

    def start_gather(c):
        idx_chunk = idx_v.at[pl.ds(c * _CROWS * _HIST, _CROWS * _HIST)]
        return pltpu.async_copy(table_hbm.at[idx_chunk], rows_v.at[c % _NBUF],
                                gsem[c % _NBUF])

    def start_writes(c):
        b = c % _NBUF
        return [
            pltpu.async_copy(
                rows_v.at[b].at[pl.ds(r * _HIST, _HIST)],
                out_hbm.at[base + c * _CROWS + r].at[pl.ds(0, _HIST),
                                                     pl.ds(0, _D)],
                osem[b])
            for r in range(_CROWS)
        ]

    gather = [None] * _NCHUNK
    write = [None] * _NCHUNK
    for c in range(_NBUF - 1):
        gather[c] = start_gather(c)
    for c in range(_NCHUNK):
        cn = c + _NBUF - 1
        if cn < _NCHUNK:
            prev = cn - _NBUF
            if prev >= 0:
                for h in write[prev]:
                    h.wait()
            gather[cn] = start_gather(cn)
        gather[c].wait()
        write[c] = start_writes(c)
    for c in range(_NCHUNK - _NBUF, _NCHUNK):
        if c >= 0:
            for h in write[c]:
                h.wait()


_TB = 256


def _tc_transpose_body(x_ref, o_ref):
    x = x_ref[...].reshape(_TB, _HIST * _D)
    o_ref[...] = jnp.transpose(x, (1, 0)).reshape(_HIST, _D, _TB)


_tc_transpose = pl.pallas_call(
    _tc_transpose_body,
    grid=(_BATCH // _TB,),
    in_specs=[pl.BlockSpec((_TB, _HIST, _D), lambda i: (i, 0, 0))],
    out_specs=pl.BlockSpec((_HIST, _D, _TB), lambda i: (0, 0, i)),
    out_shape=jax.ShapeDtypeStruct((_HIST, _D, _BATCH), jnp.float32),
)


def kernel(inputs, emb_table):
    idx = inputs.astype(jnp.int32).reshape(_NW, _ROWS_PER_W * _HIST)
    padded = _sc_gather(idx, emb_table)
    out_t = _tc_transpose(padded[:, :_HIST, :_D])
    return out_t.transpose(2, 0, 1)


# speedup vs baseline: 1.1264x; 1.0725x over previous
import functools

import jax
import jax.numpy as jnp
from jax import lax
from jax.experimental import pallas as pl
from jax.experimental.pallas import tpu as pltpu
from jax.experimental.pallas import tpu_sc as plsc

_BATCH = 4096
_HIST = 50
_HIST_PAD = 56
_D = 64
_D_PAD = 128
_NC = 2
_NS = 16
_NW = _NC * _NS
_ROWS_PER_W = _BATCH // _NW
_CROWS = 8
_NCHUNK = _ROWS_PER_W // _CROWS
_NBUF = 3

_mesh = plsc.VectorSubcoreMesh(core_axis_name="c", subcore_axis_name="s")


@functools.partial(
    pl.kernel,
    mesh=_mesh,
    out_type=jax.ShapeDtypeStruct((_BATCH, _HIST_PAD, _D_PAD), jnp.float32),
    scratch_types=[
        pltpu.VMEM((_ROWS_PER_W * _HIST,), jnp.int32),
        pltpu.VMEM((_NBUF, _CROWS * _HIST, _D), jnp.float32),
        pltpu.SemaphoreType.DMA,
        pltpu.SemaphoreType.DMA,
        pltpu.SemaphoreType.DMA,
        pltpu.SemaphoreType.DMA,
        pltpu.SemaphoreType.DMA,
        pltpu.SemaphoreType.DMA,
    ],
    compiler_params=pltpu.CompilerParams(use_tc_tiling_on_sc=False),
)
def _sc_gather(idx_hbm, table_hbm, out_hbm, idx_v, rows_v,
               g0, g1, g2, o0, o1, o2):
    wid = lax.axis_index("s") * _NC + lax.axis_index("c")
    base = wid * _ROWS_PER_W
    gsem = (g0, g1, g2)
    osem = (o0, o1, o2)
    pltpu.sync_copy(idx_hbm.at[wid], idx_v)

    def start_gather(c):
        idx_chunk = idx_v.at[pl.ds(c * _CROWS * _HIST, _CROWS * _HIST)]
        return pltpu.async_copy(table_hbm.at[idx_chunk], rows_v.at[c % _NBUF],
                                gsem[c % _NBUF])

    def start_writes(c):
        b = c % _NBUF
        return [
            pltpu.async_copy(
                rows_v.at[b].at[pl.ds(r * _HIST, _HIST)],
                out_hbm.at[base + c * _CROWS + r].at[pl.ds(0, _HIST),
                                                     pl.ds(0, _D)],
                osem[b])
            for r in range(_CROWS)
        ]

    gather = [None] * _NCHUNK
    write = [None] * _NCHUNK
    for c in range(_NBUF - 1):
        gather[c] = start_gather(c)
    for c in range(_NCHUNK):
        cn = c + _NBUF - 1
        if cn < _NCHUNK:
            prev = cn - _NBUF
            if prev >= 0:
                for h in write[prev]:
                    h.wait()
            gather[cn] = start_gather(cn)
        gather[c].wait()
        write[c] = start_writes(c)
    for c in range(_NCHUNK - _NBUF, _NCHUNK):
        if c >= 0:
            for h in write[c]:
                h.wait()


def kernel(inputs, emb_table):
    idx = inputs.astype(jnp.int32).reshape(_NW, _ROWS_PER_W * _HIST)
    padded = _sc_gather(idx, emb_table)
    return padded[:, :_HIST, :_D]
